# Initial kernel scaffold; baseline (speedup 1.0000x reference)
import jax
import jax.numpy as jnp
from jax.experimental import pallas as pl


def kernel(h, pos, edge_attr, edge_index, ew1, eb1, elg, elb, ew2, eb2, nw1, nb1, nlg, nlb, nw2, nb2, ng, nb, cw1, cb1, cw2, aw, ab, fw1, fb1, fw2, fb2):
    raise NotImplementedError("write your pallas kernel here")


# speedup vs baseline: 2.0179x; 2.0179x over previous
import functools

import jax
import jax.numpy as jnp
from jax import lax
from jax.experimental import pallas as pl
from jax.experimental.pallas import tpu as pltpu
from jax.experimental.pallas import tpu_sc as plsc

_NC = 2
_NS = 16
_NW = _NC * _NS
_CHUNK = 128


def _silu(x):
    return x * jax.nn.sigmoid(x)


def _layernorm(x, g, b):
    mu = jnp.mean(x, axis=-1, keepdims=True)
    var = jnp.mean((x - mu) ** 2, axis=-1, keepdims=True)
    return (x - mu) / jnp.sqrt(var + 1e-5) * g + b


def _sc_gather2(table, row_idx, col_idx):
    n, d = table.shape
    e = row_idx.shape[0]
    chunks = e // _CHUNK
    iters = -(-chunks // _NW)
    mesh = plsc.VectorSubcoreMesh(core_axis_name="c", subcore_axis_name="s")

    @functools.partial(
        pl.kernel, mesh=mesh,
        out_type=(jax.ShapeDtypeStruct((e, d), jnp.float32),
                  jax.ShapeDtypeStruct((e, d), jnp.float32)),
        scratch_types=[
            pltpu.VMEM((_CHUNK,), jnp.int32),
            pltpu.VMEM((_CHUNK,), jnp.int32),
            pltpu.VMEM((_CHUNK, d), jnp.float32),
            pltpu.VMEM((_CHUNK, d), jnp.float32),
            pltpu.SemaphoreType.DMA,
            pltpu.SemaphoreType.DMA,
        ],
        compiler_params=pltpu.CompilerParams(use_tc_tiling_on_sc=False),
    )
    def k(t_hbm, r_hbm, c_hbm, gr_hbm, gc_hbm, ri, ci, rb, cb, s1, s2):
        wid = lax.axis_index("s") * _NC + lax.axis_index("c")

        def body(i, _):
            cidx = wid + i * _NW

            @pl.when(cidx < chunks)
            def _():
                off = pl.multiple_of(cidx * _CHUNK, _CHUNK)
                pltpu.sync_copy(r_hbm.at[pl.ds(off, _CHUNK)], ri)
                pltpu.sync_copy(c_hbm.at[pl.ds(off, _CHUNK)], ci)
                a = pltpu.async_copy(t_hbm.at[ri], rb, s1)
                b = pltpu.async_copy(t_hbm.at[ci], cb, s2)
                a.wait()
                b.wait()
                pltpu.sync_copy(rb, gr_hbm.at[pl.ds(off, _CHUNK)])
                pltpu.sync_copy(cb, gc_hbm.at[pl.ds(off, _CHUNK)])
            return 0

        lax.fori_loop(0, iters, body, 0)

    return k(table, row_idx, col_idx)


def _sc_scatter_add(payload, row_idx, npad):
    e, d = payload.shape
    chunks = e // _CHUNK
    iters = -(-chunks // _NW)
    tpb = npad // _NS
    zeros = jnp.zeros((tpb, d), jnp.float32)
    mesh = plsc.VectorSubcoreMesh(core_axis_name="c", subcore_axis_name="s")

    @functools.partial(
        pl.kernel, mesh=mesh,
        out_type=jax.ShapeDtypeStruct((_NC * npad, d), jnp.float32),
        scratch_types=[
            pltpu.VMEM((_CHUNK,), jnp.int32),
            pltpu.VMEM((_CHUNK, d), jnp.float32),
            pltpu.VMEM_SHARED((npad, d), jnp.float32),
        ],
        compiler_params=pltpu.CompilerParams(use_tc_tiling_on_sc=False),
    )
    def k(p_hbm, r_hbm, z_hbm, out_hbm, ri, pb, acc):
        cid = lax.axis_index("c")
        sid = lax.axis_index("s")
        wid = sid * _NC + cid
        zoff = pl.multiple_of(sid * tpb, 8)
        pltpu.sync_copy(z_hbm, acc.at[pl.ds(zoff, tpb)])
        plsc.subcore_barrier()

        def body(i, _):
            cidx = wid + i * _NW

            @pl.when(cidx < chunks)
            def _():
                off = pl.multiple_of(cidx * _CHUNK, _CHUNK)
                pltpu.sync_copy(r_hbm.at[pl.ds(off, _CHUNK)], ri)
                pltpu.sync_copy(p_hbm.at[pl.ds(off, _CHUNK)], pb)
                pltpu.sync_copy(pb, acc.at[ri], add=True)
            return 0

        lax.fori_loop(0, iters, body, 0)
        plsc.subcore_barrier()
        ooff = pl.multiple_of(cid * npad + sid * tpb, 8)
        pltpu.sync_copy(acc.at[pl.ds(zoff, tpb)], out_hbm.at[pl.ds(ooff, tpb)])

    return k(payload, row_idx, zeros)


def _full(shape):
    return pl.BlockSpec(shape, lambda i: (0, 0))


def _edge_mlp(gr, gc, ea, wa, wb, wdist, wea, eb1, elg, elb, ew2, eb2,
              awr, ab, cw1, cb1, cw2r):
    e, dt = gr.shape
    h = wa.shape[0]
    be = 2000
    ed = ea.shape[1]

    def body(gr_ref, gc_ref, ea_ref, wa_ref, wb_ref, wd_ref, we_ref,
             eb1_ref, elg_ref, elb_ref, ew2_ref, eb2_ref, awr_ref, ab_ref,
             cw1_ref, cb1_ref, cw2r_ref, out_ref):
        gr_b = gr_ref[...]
        gc_b = gc_ref[...]
        hr = gr_b[:, :h]
        hc = gc_b[:, :h]
        diffp = gr_b[:, h:] - gc_b[:, h:]
        ss = jnp.sum(diffp * diffp, axis=1, keepdims=True)
        dist = jnp.sqrt(jnp.maximum(ss, 1e-10))
        pre = (jnp.dot(hr, wa_ref[...], preferred_element_type=jnp.float32)
               + jnp.dot(hc, wb_ref[...], preferred_element_type=jnp.float32)
               + jnp.dot(ea_ref[...], we_ref[...],
                         preferred_element_type=jnp.float32)
               + dist * wd_ref[...] + eb1_ref[...])
        x = _layernorm(_silu(pre), elg_ref[...], elb_ref[...])
        m = _silu(jnp.dot(x, ew2_ref[...],
                          preferred_element_type=jnp.float32) + eb2_ref[...])
        att = jax.nn.sigmoid(jnp.sum(m * awr_ref[...], axis=1, keepdims=True)
                             + ab_ref[...])
        m_att = m * att
        cwv = _silu(jnp.dot(m, cw1_ref[...],
                            preferred_element_type=jnp.float32) + cb1_ref[...])
        cws = jnp.sum(cwv * cw2r_ref[...], axis=1, keepdims=True)
        lanes = lax.broadcasted_iota(jnp.int32, (1, dt - h), 1)
        degcol = jnp.where(lanes == 3, 1.0, 0.0).astype(jnp.float32)
        tail = diffp * cws + degcol
        out_ref[...] = jnp.concatenate([m_att, tail], axis=1)

    return pl.pallas_call(
        body,
        grid=(e // be,),
        in_specs=[
            pl.BlockSpec((be, dt), lambda i: (i, 0)),
            pl.BlockSpec((be, dt), lambda i: (i, 0)),
            pl.BlockSpec((be, ed), lambda i: (i, 0)),
            _full(wa.shape), _full(wb.shape), _full(wdist.shape),
            _full(wea.shape), _full(eb1.shape), _full(elg.shape),
            _full(elb.shape), _full(ew2.shape), _full(eb2.shape),
            _full(awr.shape), _full(ab.shape), _full(cw1.shape),
            _full(cb1.shape), _full(cw2r.shape),
        ],
        out_specs=pl.BlockSpec((be, dt), lambda i: (i, 0)),
        out_shape=jax.ShapeDtypeStruct((e, dt), jnp.float32),
    )(gr, gc, ea, wa, wb, wdist, wea, eb1, elg, elb, ew2, eb2,
      awr, ab, cw1, cb1, cw2r)


def _node_mlp(h, p0, p1, pos_pad, nw1a, nw1b, nb1, nlg, nlb, nw2, nb2,
              ng, nb_, fb2r):
    n, hd = h.shape
    dt = p0.shape[1]
    pd = pos_pad.shape[1]
    bn = 2000

    def body(h_ref, p0_ref, p1_ref, pp_ref, w1a_ref, w1b_ref, nb1_ref,
             nlg_ref, nlb_ref, nw2_ref, nb2_ref, ng_ref, nb_ref, fb2_ref,
             hmid_ref, pn_ref):
        hb = h_ref[...]
        s = p0_ref[...] + p1_ref[...]
        agg = s[:, :hd]
        coord = s[:, hd:hd + 3]
        deg = s[:, hd + 3:hd + 4]
        pre = (jnp.dot(hb, w1a_ref[...], preferred_element_type=jnp.float32)
               + jnp.dot(agg, w1b_ref[...], preferred_element_type=jnp.float32)
               + nb1_ref[...])
        y = _layernorm(_silu(pre), nlg_ref[...], nlb_ref[...])
        y = jnp.dot(y, nw2_ref[...],
                    preferred_element_type=jnp.float32) + nb2_ref[...]
        hmid = _layernorm(hb + y, ng_ref[...], nb_ref[...])
        hmid_ref[...] = hmid + 0.1 * deg * fb2_ref[...]
        coord_pad = jnp.concatenate(
            [coord, jnp.zeros((coord.shape[0], pd - 3), jnp.float32)], axis=1)
        pn_ref[...] = pp_ref[...] + coord_pad

    return pl.pallas_call(
        body,
        grid=(n // bn,),
        in_specs=[
            pl.BlockSpec((bn, hd), lambda i: (i, 0)),
            pl.BlockSpec((bn, dt), lambda i: (i, 0)),
            pl.BlockSpec((bn, dt), lambda i: (i, 0)),
            pl.BlockSpec((bn, pd), lambda i: (i, 0)),
            _full(nw1a.shape), _full(nw1b.shape), _full(nb1.shape),
            _full(nlg.shape), _full(nlb.shape), _full(nw2.shape),
            _full(nb2.shape), _full(ng.shape), _full(nb_.shape),
            _full(fb2r.shape),
        ],
        out_specs=[
            pl.BlockSpec((bn, hd), lambda i: (i, 0)),
            pl.BlockSpec((bn, pd), lambda i: (i, 0)),
        ],
        out_shape=[
            jax.ShapeDtypeStruct((n, hd), jnp.float32),
            jax.ShapeDtypeStruct((n, pd), jnp.float32),
        ],
    )(h, p0, p1, pos_pad, nw1a, nw1b, nb1, nlg, nlb, nw2, nb2, ng, nb_, fb2r)


def _edge_dist_stage(pnr, pnc, fw1r, fb1r):
    e, pd = pnr.shape
    hd = fw1r.shape[1]
    be = 4000

    def body(pnr_ref, pnc_ref, fw1_ref, fb1_ref, out_ref):
        d = pnr_ref[...] - pnc_ref[...]
        ss = jnp.sum(d * d, axis=1, keepdims=True)
        dist = jnp.sqrt(jnp.maximum(ss, 1e-10))
        out_ref[...] = _silu(dist * fw1_ref[...] + fb1_ref[...])

    return pl.pallas_call(
        body,
        grid=(e // be,),
        in_specs=[
            pl.BlockSpec((be, pd), lambda i: (i, 0)),
            pl.BlockSpec((be, pd), lambda i: (i, 0)),
            _full(fw1r.shape), _full(fb1r.shape),
        ],
        out_specs=pl.BlockSpec((be, hd), lambda i: (i, 0)),
        out_shape=jax.ShapeDtypeStruct((e, hd), jnp.float32),
    )(pnr, pnc, fw1r, fb1r)


def _final_stage(hmid, q0, q1, fw2):
    n, hd = hmid.shape
    bn = 2000

    def body(hm_ref, q0_ref, q1_ref, fw2_ref, out_ref):
        s = q0_ref[...] + q1_ref[...]
        out_ref[...] = hm_ref[...] + 0.1 * jnp.dot(
            s, fw2_ref[...], preferred_element_type=jnp.float32)

    return pl.pallas_call(
        body,
        grid=(n // bn,),
        in_specs=[
            pl.BlockSpec((bn, hd), lambda i: (i, 0)),
            pl.BlockSpec((bn, hd), lambda i: (i, 0)),
            pl.BlockSpec((bn, hd), lambda i: (i, 0)),
            _full(fw2.shape),
        ],
        out_specs=pl.BlockSpec((bn, hd), lambda i: (i, 0)),
        out_shape=jax.ShapeDtypeStruct((n, hd), jnp.float32),
    )(hmid, q0, q1, fw2)


def kernel(h, pos, edge_attr, edge_index, ew1, eb1, elg, elb, ew2, eb2,
           nw1, nb1, nlg, nlb, nw2, nb2, ng, nb, cw1, cb1, cw2, aw, ab,
           fw1, fb1, fw2, fb2):
    n, hd = h.shape
    e = edge_index.shape[1]
    row = edge_index[0]
    col = edge_index[1]

    pos_pad = jnp.pad(pos, ((0, 0), (0, 16 - pos.shape[1])))
    table = jnp.concatenate([h, pos_pad], axis=1)

    gr, gc = _sc_gather2(table, row, col)

    r1 = lambda v: v.reshape(1, -1)
    payload = _edge_mlp(
        gr, gc, edge_attr,
        ew1[:hd], ew1[hd:2 * hd], ew1[2 * hd:2 * hd + 1], ew1[2 * hd + 1:],
        r1(eb1), r1(elg), r1(elb), ew2, r1(eb2),
        aw.reshape(1, -1), ab.reshape(1, 1), cw1, r1(cb1), cw2.reshape(1, -1))

    tpb = (-(-n // _NS) + 7) // 8 * 8
    npad = tpb * _NS
    part = _sc_scatter_add(payload, row, npad)
    p0 = part[:n]
    p1 = part[npad:npad + n]

    hmid, pn = _node_mlp(
        h, p0, p1, pos_pad, nw1[:hd], nw1[hd:], r1(nb1), r1(nlg), r1(nlb),
        nw2, r1(nb2), r1(ng), r1(nb), r1(fb2))

    pnr, pnc = _sc_gather2(pn, row, col)
    s_e = _edge_dist_stage(pnr, pnc, fw1, r1(fb1))

    part2 = _sc_scatter_add(s_e, row, npad)
    q0 = part2[:n]
    q1 = part2[npad:npad + n]

    h_new = _final_stage(hmid, q0, q1, fw2)
    pos_new = pn[:, :pos.shape[1]]
    return (h_new, pos_new)
